# baseline (speedup 1.0000x reference)
import math

import jax
import jax.numpy as jnp
from jax import lax
from jax.experimental import pallas as pl
from jax.experimental.pallas import tpu as pltpu
from jax.experimental.pallas import tpu_sc as plsc

NC = 2
NS = 16
NW = NC * NS
LANES = 16
NBUF = 4
LEAD = 2
SPLIT = 104


def _sc_embedding_lookup(tok, table, scale):
    bsz, seq = tok.shape
    d = table.shape[1]
    rows_per_w = bsz // NW
    mesh = plsc.VectorSubcoreMesh(core_axis_name="c", subcore_axis_name="s")

    @pl.kernel(
        out_type=jax.ShapeDtypeStruct((bsz, seq, d), jnp.float32),
        mesh=mesh,
        compiler_params=pltpu.CompilerParams(use_tc_tiling_on_sc=False),
        scratch_types=[
            pltpu.VMEM((rows_per_w, seq), jnp.int32),
            pltpu.VMEM((NBUF, seq, d), jnp.float32),
            pltpu.SemaphoreType.DMA((NBUF,)),
            pltpu.SemaphoreType.DMA((NBUF,)),
        ],
    )
    def k(tok_hbm, table_hbm, out_hbm, idx_v, gbuf, gsem, ssem):
        wid = lax.axis_index("c") * NS + lax.axis_index("s")
        row0 = wid * rows_per_w

        pltpu.sync_copy(tok_hbm.at[pl.ds(row0, rows_per_w)], idx_v)

        def gather_copies(i, b):
            return (
                pltpu.make_async_copy(
                    table_hbm.at[idx_v.at[i, pl.ds(0, SPLIT)]],
                    gbuf.at[b, pl.ds(0, SPLIT)], gsem.at[b]),
                pltpu.make_async_copy(
                    table_hbm.at[idx_v.at[i, pl.ds(SPLIT, seq - SPLIT)]],
                    gbuf.at[b, pl.ds(SPLIT, seq - SPLIT)], gsem.at[b]),
            )

        def store_copy(i, b):
            return pltpu.make_async_copy(
                gbuf.at[b], out_hbm.at[row0 + i], ssem.at[b])

        def issue_gather(i, b):
            for cp in gather_copies(i, b):
                cp.start()

        def wait_gather(i, b):
            for cp in gather_copies(i, b):
                cp.wait()

        def scale_rows(b):
            g = gbuf.at[b]

            @pl.loop(0, seq, step=8)
            def _(r):
                for dr in range(8):
                    for c in range(d // LANES):
                        sl = (pl.ds(r + dr, 1), pl.ds(c * LANES, LANES))
                        g.at[sl][...] = g.at[sl][...] * scale

        def process(i, k_slot, refill):
            wait_gather(i, k_slot)
            scale_rows(k_slot)
            store_copy(i, k_slot).start()
            if refill:
                nxt = (k_slot + LEAD) % NBUF
                store_copy(i - LEAD, nxt).wait()
                issue_gather(i + LEAD, nxt)

        for b in range(NBUF):
            issue_gather(b, b)
        for kk in range(NBUF):
            process(kk, kk, refill=(kk >= LEAD))

        @pl.loop(1, rows_per_w // NBUF - 1)
        def _(grp):
            i0 = grp * NBUF
            for kk in range(NBUF):
                process(i0 + kk, kk, refill=True)

        i0 = rows_per_w - NBUF
        for kk in range(NBUF):
            process(i0 + kk, kk, refill=(kk < LEAD))

        for kk in range(NBUF):
            store_copy(i0 + kk, kk).wait()

    return k(tok, table)


def kernel(token_ids, embedding_table):
    bsz, seq = token_ids.shape
    d = embedding_table.shape[1]
    assert bsz % NW == 0 and (bsz // NW) % NBUF == 0 and d % LANES == 0
    scale = math.sqrt(d)
    return _sc_embedding_lookup(
        token_ids.astype(jnp.int32), embedding_table, scale)
